# Initial kernel scaffold; baseline (speedup 1.0000x reference)
import jax
import jax.numpy as jnp
from jax.experimental import pallas as pl


def kernel(scores):
    raise NotImplementedError("write your pallas kernel here")


# speedup vs baseline: 20.9513x; 20.9513x over previous
import jax
import jax.numpy as jnp
from jax import lax
from jax.experimental import pallas as pl

_TOPK = 2048


def _topk_mean_body(x_ref, o_ref):
    x = x_ref[...]
    u = lax.bitcast_convert_type(x, jnp.uint32)
    neg = u >= jnp.uint32(0x80000000)
    su = jnp.where(neg, ~u, u | jnp.uint32(0x80000000))

    def step(i, prefix):
        bit = jnp.uint32(1) << (jnp.uint32(31) - i.astype(jnp.uint32))
        cand = prefix | bit
        ge = su >= cand[:, None, None]
        cnt = jnp.sum(ge.astype(jnp.int32), axis=(1, 2))
        return jnp.where(cnt >= _TOPK, cand, prefix)

    t = lax.fori_loop(0, 32, step, jnp.zeros((64,), jnp.uint32))
    gt = su > t[:, None, None]
    cnt_gt = jnp.sum(gt.astype(jnp.int32), axis=(1, 2))
    sum_gt = jnp.sum(jnp.where(gt, x, 0.0), axis=(1, 2))
    tv_u = jnp.where(t >= jnp.uint32(0x80000000), t ^ jnp.uint32(0x80000000), ~t)
    tval = lax.bitcast_convert_type(tv_u, jnp.float32)
    total = sum_gt + (jnp.float32(_TOPK) - cnt_gt.astype(jnp.float32)) * tval
    mean = total * jnp.float32(1.0 / _TOPK)
    o_ref[...] = jnp.broadcast_to(mean[:, None], (64, 128))


def kernel(scores):
    x = scores.reshape(64, 256, 128)
    out = pl.pallas_call(
        _topk_mean_body,
        out_shape=jax.ShapeDtypeStruct((64, 128), jnp.float32),
    )(x)
    return out[:, 0]
